# baseline (speedup 1.0000x reference)
import functools

import jax
import jax.numpy as jnp
from jax import lax
from jax.experimental import pallas as pl
from jax.experimental.pallas import tpu as pltpu

E = 8
TOPK = 2
BT = 256


def _moe_dense_kernel(x_ref, gate_ref, w1_ref, b1_ref, w2_ref, b2_ref,
                      out_ref, comb_ref):
    t = pl.program_id(0)
    e = pl.program_id(1)
    f = pl.program_id(2)

    @pl.when(jnp.logical_and(e == 0, f == 0))
    def _router():
        xb = x_ref[...]
        s = lax.dot_general(xb, gate_ref[...],
                            (((1,), (1,)), ((), ())),
                            preferred_element_type=jnp.float32)
        idx1 = jnp.argmax(s, axis=1)
        cols = lax.broadcasted_iota(jnp.int32, s.shape, 1)
        oh1 = (cols == idx1[:, None])
        m1 = jnp.max(s, axis=1, keepdims=True)
        s2 = jnp.where(oh1, -jnp.inf, s)
        idx2 = jnp.argmax(s2, axis=1)
        oh2 = (cols == idx2[:, None])
        m2 = jnp.max(s2, axis=1, keepdims=True)
        e2 = jnp.exp(m2 - m1)
        z = 1.0 + e2
        p1 = 1.0 / z
        p2 = e2 / z
        comb_ref[...] = jnp.where(oh1, p1, 0.0) + jnp.where(oh2, p2, 0.0)
        out_ref[...] = jnp.zeros_like(out_ref)

    xb = x_ref[...].astype(jnp.bfloat16)
    w1b = w1_ref[0].astype(jnp.bfloat16)
    h = lax.dot_general(xb, w1b, (((1,), (1,)), ((), ())),
                        preferred_element_type=jnp.float32)
    h = h + b1_ref[0]
    h = 0.5 * h * (1.0 + lax.erf(h * 0.7071067811865476))
    w2b = w2_ref[0].astype(jnp.bfloat16)
    y = lax.dot_general(h.astype(jnp.bfloat16), w2b, (((1,), (1,)), ((), ())),
                        preferred_element_type=jnp.float32)
    comb = comb_ref[...]
    cols_e = lax.broadcasted_iota(jnp.int32, comb.shape, 1)
    ce = jnp.sum(jnp.where(cols_e == e, comb, 0.0), axis=1, keepdims=True)

    @pl.when(f == 0)
    def _bias2():
        out_ref[...] += ce * b2_ref[0]

    out_ref[...] += ce * y


def kernel(x, gate_w, w1, b1, w2, b2):
    b, s, d = x.shape
    xf = x.reshape(-1, d)
    T = xf.shape[0]
    n_exp, f_dim = w1.shape[0], w1.shape[1]
    FB = 1024
    nf = f_dim // FB
    nt = T // BT

    b1r = b1.reshape(n_exp, 1, f_dim)
    b2r = b2.reshape(n_exp, 1, d)

    out = pl.pallas_call(
        _moe_dense_kernel,
        grid=(nt, n_exp, nf),
        in_specs=[
            pl.BlockSpec((BT, d), lambda t, e, f: (t, 0)),
            pl.BlockSpec((n_exp, d), lambda t, e, f: (0, 0)),
            pl.BlockSpec((1, FB, d), lambda t, e, f: (e, f, 0)),
            pl.BlockSpec((1, 1, FB), lambda t, e, f: (e, 0, f)),
            pl.BlockSpec((1, d, FB), lambda t, e, f: (e, 0, f)),
            pl.BlockSpec((1, 1, d), lambda t, e, f: (e, 0, 0)),
        ],
        out_specs=pl.BlockSpec((BT, d), lambda t, e, f: (t, 0)),
        out_shape=jax.ShapeDtypeStruct((T, d), jnp.float32),
        scratch_shapes=[pltpu.VMEM((BT, n_exp), jnp.float32)],
    )(xf, gate_w, w1, b1r, w2, b2r)
    return out.reshape(b, s, d)


# speedup vs baseline: 1.2925x; 1.2925x over previous
import jax
import jax.numpy as jnp
from jax import lax
from jax.experimental import pallas as pl
from jax.experimental.pallas import tpu as pltpu

BT = 256


def _moe_dense_kernel(x_ref, gate_ref, w1_ref, b1_ref, w2_ref, b2_ref,
                      out_ref, comb_ref):
    e = pl.program_id(0)
    f = pl.program_id(1)
    t = pl.program_id(2)
    n_exp = gate_ref.shape[0]

    xb = x_ref[...]

    @pl.when(jnp.logical_and(e == 0, f == 0))
    def _router():
        s = lax.dot_general(xb, gate_ref[...],
                            (((1,), (1,)), ((), ())),
                            preferred_element_type=jnp.float32)
        idx1 = jnp.argmax(s, axis=1)
        cols = lax.broadcasted_iota(jnp.int32, s.shape, 1)
        oh1 = (cols == idx1[:, None])
        m1 = jnp.max(s, axis=1, keepdims=True)
        s2 = jnp.where(oh1, -jnp.inf, s)
        idx2 = jnp.argmax(s2, axis=1)
        oh2 = (cols == idx2[:, None])
        m2 = jnp.max(s2, axis=1, keepdims=True)
        e2 = jnp.exp(m2 - m1)
        z = 1.0 + e2
        comb_ref[pl.ds(t * BT, BT), :] = (
            jnp.where(oh1, 1.0 / z, 0.0) + jnp.where(oh2, e2 / z, 0.0))
        out_ref[pl.ds(t * BT, BT), :] = jnp.zeros((BT, out_ref.shape[1]),
                                                  jnp.float32)

    w1b = w1_ref[0].astype(jnp.bfloat16)
    h = lax.dot_general(xb.astype(jnp.bfloat16), w1b,
                        (((1,), (1,)), ((), ())),
                        preferred_element_type=jnp.float32)
    h = h + b1_ref[0]
    h = 0.5 * h * (1.0 + lax.erf(h * 0.7071067811865476))
    w2b = w2_ref[0].astype(jnp.bfloat16)
    y = lax.dot_general(h.astype(jnp.bfloat16), w2b,
                        (((1,), (1,)), ((), ())),
                        preferred_element_type=jnp.float32)

    comb = comb_ref[pl.ds(t * BT, BT), :]
    cols_e = lax.broadcasted_iota(jnp.int32, comb.shape, 1)
    ce = jnp.sum(jnp.where(cols_e == e, comb, 0.0), axis=1, keepdims=True)

    @pl.when(f == 0)
    def _bias2():
        out_ref[pl.ds(t * BT, BT), :] += ce * b2_ref[0]

    out_ref[pl.ds(t * BT, BT), :] += ce * y


def kernel(x, gate_w, w1, b1, w2, b2):
    b, s, d = x.shape
    xf = x.reshape(-1, d)
    T = xf.shape[0]
    n_exp, f_dim = w1.shape[0], w1.shape[1]
    FB = 1024
    nf = f_dim // FB
    nt = T // BT

    b1r = b1.reshape(n_exp, 1, f_dim)
    b2r = b2.reshape(n_exp, 1, d)

    out = pl.pallas_call(
        _moe_dense_kernel,
        grid=(n_exp, nf, nt),
        in_specs=[
            pl.BlockSpec((BT, d), lambda e, f, t: (t, 0)),
            pl.BlockSpec((n_exp, d), lambda e, f, t: (0, 0)),
            pl.BlockSpec((1, FB, d), lambda e, f, t: (e, f, 0)),
            pl.BlockSpec((1, 1, FB), lambda e, f, t: (e, 0, f)),
            pl.BlockSpec((1, d, FB), lambda e, f, t: (e, 0, f)),
            pl.BlockSpec((1, 1, d), lambda e, f, t: (e, 0, 0)),
        ],
        out_specs=pl.BlockSpec((T, d), lambda e, f, t: (0, 0)),
        out_shape=jax.ShapeDtypeStruct((T, d), jnp.float32),
        scratch_shapes=[pltpu.VMEM((T, n_exp), jnp.float32)],
    )(xf, gate_w, w1, b1r, w2, b2r)
    return out.reshape(b, s, d)


# speedup vs baseline: 1.9309x; 1.4939x over previous
import functools

import jax
import jax.numpy as jnp
from jax import lax
from jax.experimental import pallas as pl
from jax.experimental.pallas import tpu as pltpu
from jax.experimental.pallas import tpu_sc as plsc

NEXP = 8
BTS = 256
NROWS = 5888
NBLK = NROWS // BTS


def _router_kernel(x_ref, gate_ref, d0_ref, d1_ref, p0_ref, p1_ref, be_ref):
    x = x_ref[...]
    T = x.shape[0]
    s = lax.dot_general(x, gate_ref[...], (((1,), (1,)), ((), ())),
                        preferred_element_type=jnp.float32)
    cols = lax.broadcasted_iota(jnp.int32, s.shape, 1)
    idx1 = jnp.argmax(s, axis=1)
    oh1 = (cols == idx1[:, None])
    m1 = jnp.max(s, axis=1, keepdims=True)
    s2 = jnp.where(oh1, -jnp.inf, s)
    idx2 = jnp.argmax(s2, axis=1)
    oh2 = (cols == idx2[:, None])
    m2 = jnp.max(s2, axis=1, keepdims=True)
    e2 = jnp.exp(m2 - m1)
    z = 1.0 + e2
    p0_ref[...] = 1.0 / z
    p1_ref[...] = e2 / z

    oh1f = oh1.astype(jnp.float32)
    oh2f = oh2.astype(jnp.float32)
    ohp = jnp.concatenate([oh1f, oh2f], axis=0)

    n = 2 * T
    acc = ohp
    k = 1
    while k < n:
        shifted = jnp.concatenate(
            [jnp.zeros((k, NEXP), jnp.float32), acc[:-k]], axis=0)
        acc = acc + shifted
        k *= 2
    excl = acc - ohp
    counts = acc[n - 1:n, :]

    pc = jnp.floor((counts + (BTS - 1)) * (1.0 / BTS)) * BTS
    end = pc
    k = 1
    while k < NEXP:
        end = end + jnp.concatenate(
            [jnp.zeros((1, k), jnp.float32), end[:, :-k]], axis=1)
        k *= 2
    off = end - pc

    offb = jnp.broadcast_to(off, (T, NEXP))
    d0 = jnp.sum(oh1f * (excl[:T] + offb), axis=1, keepdims=True)
    d1 = jnp.sum(oh2f * (excl[T:] + offb), axis=1, keepdims=True)
    d0_ref[...] = d0.astype(jnp.int32)
    d1_ref[...] = d1.astype(jnp.int32)

    jrow = (lax.broadcasted_iota(jnp.int32, (NBLK, NEXP), 0)
            .astype(jnp.float32) * float(BTS))
    endb = jnp.broadcast_to(end, (NBLK, NEXP))
    be = jnp.sum((jrow >= endb).astype(jnp.float32), axis=1, keepdims=True)
    be_ref[...] = jnp.clip(be, 0.0, float(NEXP - 1)).astype(jnp.int32)


def _route(xf, gate_w):
    T, d = xf.shape
    return pl.pallas_call(
        _router_kernel,
        grid=(1,),
        in_specs=[
            pl.BlockSpec((T, d), lambda i: (0, 0)),
            pl.BlockSpec((NEXP, d), lambda i: (0, 0)),
        ],
        out_specs=[
            pl.BlockSpec((T, 1), lambda i: (0, 0)),
            pl.BlockSpec((T, 1), lambda i: (0, 0)),
            pl.BlockSpec((T, 1), lambda i: (0, 0)),
            pl.BlockSpec((T, 1), lambda i: (0, 0)),
            pl.BlockSpec((NBLK, 1), lambda i: (0, 0)),
        ],
        out_shape=[
            jax.ShapeDtypeStruct((T, 1), jnp.int32),
            jax.ShapeDtypeStruct((T, 1), jnp.int32),
            jax.ShapeDtypeStruct((T, 1), jnp.float32),
            jax.ShapeDtypeStruct((T, 1), jnp.float32),
            jax.ShapeDtypeStruct((NBLK, 1), jnp.int32),
        ],
    )(xf, gate_w)


def _sc_scatter(xf, dest2):
    T, d = xf.shape
    info = plsc.get_sparse_core_info()
    nc, ns = info.num_cores, info.num_subcores
    nw = nc * ns
    rows_per_w = 2 * T // nw

    @functools.partial(
        pl.kernel,
        mesh=plsc.VectorSubcoreMesh(core_axis_name="c", subcore_axis_name="s"),
        out_type=jax.ShapeDtypeStruct((NROWS, d), jnp.float32),
        scratch_types=[
            pltpu.VMEM((rows_per_w,), jnp.int32),
            pltpu.VMEM((rows_per_w, d), jnp.float32),
            pltpu.SemaphoreType.DMA,
        ],
    )
    def k(x_hbm, dest_hbm, xs_hbm, idx_v, rows_v, sem):
        wid = lax.axis_index("s") * nc + lax.axis_index("c")
        g = wid // ns
        j = wid % ns
        base = j * rows_per_w
        pltpu.sync_copy(dest_hbm.at[g, pl.ds(base, rows_per_w)], idx_v)
        pltpu.sync_copy(x_hbm.at[pl.ds(base, rows_per_w)], rows_v)
        pltpu.async_copy(rows_v, xs_hbm.at[idx_v], sem).wait()

    return k(xf, dest2)


def _sc_gather(ys, dest2):
    _, d = ys.shape
    info = plsc.get_sparse_core_info()
    nc, ns = info.num_cores, info.num_subcores
    nw = nc * ns
    T = dest2.shape[1]
    rows_per_w = 2 * T // nw

    @functools.partial(
        pl.kernel,
        mesh=plsc.VectorSubcoreMesh(core_axis_name="c", subcore_axis_name="s"),
        out_type=jax.ShapeDtypeStruct((2, T, d), jnp.float32),
        scratch_types=[
            pltpu.VMEM((rows_per_w,), jnp.int32),
            pltpu.VMEM((rows_per_w, d), jnp.float32),
            pltpu.SemaphoreType.DMA,
        ],
    )
    def k(ys_hbm, dest_hbm, yp_hbm, idx_v, rows_v, sem):
        wid = lax.axis_index("s") * nc + lax.axis_index("c")
        g = wid // ns
        j = wid % ns
        base = j * rows_per_w
        pltpu.sync_copy(dest_hbm.at[g, pl.ds(base, rows_per_w)], idx_v)
        pltpu.async_copy(ys_hbm.at[idx_v], rows_v, sem).wait()
        pltpu.sync_copy(rows_v, yp_hbm.at[g, pl.ds(base, rows_per_w)])

    return k(ys, dest2)


def _ffn_kernel(be_ref, xs_ref, w1_ref, b1_ref, w2_ref, b2_ref, out_ref):
    f = pl.program_id(0)
    b = pl.program_id(1)
    nf = pl.num_programs(0)

    xb = xs_ref[...]
    w1c = w1_ref[0]
    h = lax.dot_general(xb, w1c, (((1,), (1,)), ((), ())),
                        preferred_element_type=jnp.float32)
    h = h + b1_ref[0]
    h = 0.5 * h * (1.0 + lax.erf(h * 0.7071067811865476))
    w2c = w2_ref[0]
    y = lax.dot_general(h, w2c, (((1,), (1,)), ((), ())),
                        preferred_element_type=jnp.float32)

    @pl.when(f == 0)
    def _init():
        out_ref[pl.ds(b * BTS, BTS), :] = y + b2_ref[0]

    @pl.when(f != 0)
    def _acc():
        out_ref[pl.ds(b * BTS, BTS), :] += y


def _ffn(xs, be, w1, b1r, w2, b2r):
    d = xs.shape[1]
    f_dim = w1.shape[1]
    FC = 768
    nf = f_dim // FC
    grid_spec = pltpu.PrefetchScalarGridSpec(
        num_scalar_prefetch=1,
        grid=(nf, NBLK),
        in_specs=[
            pl.BlockSpec((BTS, d), lambda f, b, be: (b, 0)),
            pl.BlockSpec((1, FC, d), lambda f, b, be: (be[b], f, 0)),
            pl.BlockSpec((1, 1, FC), lambda f, b, be: (be[b], 0, f)),
            pl.BlockSpec((1, d, FC), lambda f, b, be: (be[b], 0, f)),
            pl.BlockSpec((1, 1, d), lambda f, b, be: (be[b], 0, 0)),
        ],
        out_specs=pl.BlockSpec((NROWS, d), lambda f, b, be: (0, 0)),
    )
    return pl.pallas_call(
        _ffn_kernel,
        grid_spec=grid_spec,
        out_shape=jax.ShapeDtypeStruct((NROWS, d), jnp.float32),
    )(be, xs, w1, b1r, w2, b2r)


def _combine_kernel(y0_ref, y1_ref, p0_ref, p1_ref, out_ref):
    out_ref[...] = p0_ref[...] * y0_ref[0] + p1_ref[...] * y1_ref[0]


def _combine(yp, p0, p1):
    _, T, d = yp.shape
    BT = 256
    return pl.pallas_call(
        _combine_kernel,
        grid=(T // BT,),
        in_specs=[
            pl.BlockSpec((1, BT, d), lambda t: (0, t, 0)),
            pl.BlockSpec((1, BT, d), lambda t: (1, t, 0)),
            pl.BlockSpec((BT, 1), lambda t: (t, 0)),
            pl.BlockSpec((BT, 1), lambda t: (t, 0)),
        ],
        out_specs=pl.BlockSpec((BT, d), lambda t: (t, 0)),
        out_shape=jax.ShapeDtypeStruct((T, d), jnp.float32),
    )(yp, yp, p0, p1)


def kernel(x, gate_w, w1, b1, w2, b2):
    b, s, d = x.shape
    xf = x.reshape(-1, d)
    n_exp, f_dim = w1.shape[0], w1.shape[1]

    d0, d1, p0, p1, be = _route(xf, gate_w)
    dest2 = jnp.concatenate([d0.reshape(1, -1), d1.reshape(1, -1)], axis=0)
    be1 = be.reshape(-1)

    xs = _sc_scatter(xf, dest2)

    b1r = b1.reshape(n_exp, 1, f_dim)
    b2r = b2.reshape(n_exp, 1, d)
    ys = _ffn(xs, be1, w1, b1r, w2, b2r)

    yp = _sc_gather(ys, dest2)
    out = _combine(yp, p0, p1)
    return out.reshape(b, s, d)


# speedup vs baseline: 2.6512x; 1.3731x over previous
import functools

import jax
import jax.numpy as jnp
from jax import lax
from jax.experimental import pallas as pl
from jax.experimental.pallas import tpu as pltpu
from jax.experimental.pallas import tpu_sc as plsc

NEXP = 8
BTS = 256
NROWS = 5888
NBLK = NROWS // BTS


def _router_kernel(x_ref, gate_ref, d0_ref, d1_ref, p0_ref, p1_ref, be_ref):
    x = x_ref[...]
    T = x.shape[0]
    s = lax.dot_general(x, gate_ref[...], (((1,), (1,)), ((), ())),
                        preferred_element_type=jnp.float32)
    cols = lax.broadcasted_iota(jnp.int32, s.shape, 1)
    idx1 = jnp.argmax(s, axis=1)
    oh1 = (cols == idx1[:, None])
    m1 = jnp.max(s, axis=1, keepdims=True)
    s2 = jnp.where(oh1, -jnp.inf, s)
    idx2 = jnp.argmax(s2, axis=1)
    oh2 = (cols == idx2[:, None])
    m2 = jnp.max(s2, axis=1, keepdims=True)
    e2 = jnp.exp(m2 - m1)
    z = 1.0 + e2
    p0_ref[...] = 1.0 / z
    p1_ref[...] = e2 / z

    oh1f = oh1.astype(jnp.float32)
    oh2f = oh2.astype(jnp.float32)
    ohp = jnp.concatenate([oh1f, oh2f], axis=0)

    n = 2 * T
    acc = ohp
    k = 1
    while k < n:
        shifted = jnp.concatenate(
            [jnp.zeros((k, NEXP), jnp.float32), acc[:-k]], axis=0)
        acc = acc + shifted
        k *= 2
    excl = acc - ohp
    counts = acc[n - 1:n, :]

    pc = jnp.floor((counts + (BTS - 1)) * (1.0 / BTS)) * BTS
    end = pc
    k = 1
    while k < NEXP:
        end = end + jnp.concatenate(
            [jnp.zeros((1, k), jnp.float32), end[:, :-k]], axis=1)
        k *= 2
    off = end - pc

    offb = jnp.broadcast_to(off, (T, NEXP))
    d0 = jnp.sum(oh1f * (excl[:T] + offb), axis=1, keepdims=True)
    d1 = jnp.sum(oh2f * (excl[T:] + offb), axis=1, keepdims=True)
    d0_ref[...] = d0.astype(jnp.int32)
    d1_ref[...] = d1.astype(jnp.int32)

    jrow = (lax.broadcasted_iota(jnp.int32, (NBLK, NEXP), 0)
            .astype(jnp.float32) * float(BTS))
    endb = jnp.broadcast_to(end, (NBLK, NEXP))
    be = jnp.sum((jrow >= endb).astype(jnp.float32), axis=1, keepdims=True)
    be_ref[...] = jnp.clip(be, 0.0, float(NEXP - 1)).astype(jnp.int32)


def _route(xf, gate_w):
    T, d = xf.shape
    return pl.pallas_call(
        _router_kernel,
        grid=(1,),
        in_specs=[
            pl.BlockSpec((T, d), lambda i: (0, 0)),
            pl.BlockSpec((NEXP, d), lambda i: (0, 0)),
        ],
        out_specs=[
            pl.BlockSpec((T, 1), lambda i: (0, 0)),
            pl.BlockSpec((T, 1), lambda i: (0, 0)),
            pl.BlockSpec((T, 1), lambda i: (0, 0)),
            pl.BlockSpec((T, 1), lambda i: (0, 0)),
            pl.BlockSpec((NBLK, 1), lambda i: (0, 0)),
        ],
        out_shape=[
            jax.ShapeDtypeStruct((T, 1), jnp.int32),
            jax.ShapeDtypeStruct((T, 1), jnp.int32),
            jax.ShapeDtypeStruct((T, 1), jnp.float32),
            jax.ShapeDtypeStruct((T, 1), jnp.float32),
            jax.ShapeDtypeStruct((NBLK, 1), jnp.int32),
        ],
    )(xf, gate_w)


def _sc_scatter(xf, d0, d1):
    T, d = xf.shape
    info = plsc.get_sparse_core_info()
    nc, ns = info.num_cores, info.num_subcores
    nw = nc * ns
    tok_per_w = T // nw

    @functools.partial(
        pl.kernel,
        mesh=plsc.VectorSubcoreMesh(core_axis_name="c", subcore_axis_name="s"),
        out_type=jax.ShapeDtypeStruct((NROWS, d), jnp.float32),
        scratch_types=[
            pltpu.VMEM((tok_per_w,), jnp.int32),
            pltpu.VMEM((tok_per_w,), jnp.int32),
            pltpu.VMEM((tok_per_w, d), jnp.float32),
            pltpu.SemaphoreType.DMA,
            pltpu.SemaphoreType.DMA,
        ],
    )
    def k(x_hbm, d0_hbm, d1_hbm, xs_hbm, idx0_v, idx1_v, rows_v, sem0, sem1):
        wid = lax.axis_index("s") * nc + lax.axis_index("c")
        base = wid * tok_per_w
        pltpu.sync_copy(d0_hbm.at[pl.ds(base, tok_per_w)], idx0_v)
        pltpu.sync_copy(d1_hbm.at[pl.ds(base, tok_per_w)], idx1_v)
        pltpu.sync_copy(x_hbm.at[pl.ds(base, tok_per_w)], rows_v)
        c0 = pltpu.async_copy(rows_v, xs_hbm.at[idx0_v], sem0)
        c1 = pltpu.async_copy(rows_v, xs_hbm.at[idx1_v], sem1)
        c0.wait()
        c1.wait()

    return k(xf, d0, d1)


def _sc_gather(ys, d0, d1):
    _, d = ys.shape
    T = d0.shape[0]
    info = plsc.get_sparse_core_info()
    nc, ns = info.num_cores, info.num_subcores
    nw = nc * ns
    tok_per_w = T // nw

    @functools.partial(
        pl.kernel,
        mesh=plsc.VectorSubcoreMesh(core_axis_name="c", subcore_axis_name="s"),
        out_type=jax.ShapeDtypeStruct((2, T, d), jnp.float32),
        scratch_types=[
            pltpu.VMEM((tok_per_w,), jnp.int32),
            pltpu.VMEM((tok_per_w,), jnp.int32),
            pltpu.VMEM((tok_per_w, d), jnp.float32),
            pltpu.VMEM((tok_per_w, d), jnp.float32),
            pltpu.SemaphoreType.DMA,
            pltpu.SemaphoreType.DMA,
        ],
    )
    def k(ys_hbm, d0_hbm, d1_hbm, yp_hbm, idx0_v, idx1_v, r0_v, r1_v,
          sem0, sem1):
        wid = lax.axis_index("s") * nc + lax.axis_index("c")
        base = wid * tok_per_w
        pltpu.sync_copy(d0_hbm.at[pl.ds(base, tok_per_w)], idx0_v)
        pltpu.sync_copy(d1_hbm.at[pl.ds(base, tok_per_w)], idx1_v)
        c0 = pltpu.async_copy(ys_hbm.at[idx0_v], r0_v, sem0)
        c1 = pltpu.async_copy(ys_hbm.at[idx1_v], r1_v, sem1)
        c0.wait()
        c1.wait()
        pltpu.sync_copy(r0_v, yp_hbm.at[0, pl.ds(base, tok_per_w)])
        pltpu.sync_copy(r1_v, yp_hbm.at[1, pl.ds(base, tok_per_w)])

    return k(ys, d0, d1)


def _ffn_kernel(be_ref, xs_ref, w1_ref, b1_ref, w2_ref, b2_ref, out_ref):
    xb = xs_ref[...]
    w1c = w1_ref[0]
    h = lax.dot_general(xb, w1c, (((1,), (1,)), ((), ())),
                        preferred_element_type=jnp.float32)
    h = h + b1_ref[0]
    h = 0.5 * h * (1.0 + lax.erf(h * 0.7071067811865476))
    w2c = w2_ref[0]
    y = lax.dot_general(h, w2c, (((1,), (1,)), ((), ())),
                        preferred_element_type=jnp.float32)
    out_ref[...] = y + b2_ref[0]


def _ffn(xs, be, w1, b1r, w2, b2r):
    d = xs.shape[1]
    f_dim = w1.shape[1]
    grid_spec = pltpu.PrefetchScalarGridSpec(
        num_scalar_prefetch=1,
        grid=(NBLK,),
        in_specs=[
            pl.BlockSpec((BTS, d), lambda b, be: (b, 0)),
            pl.BlockSpec((1, f_dim, d), lambda b, be: (be[b], 0, 0)),
            pl.BlockSpec((1, 1, f_dim), lambda b, be: (be[b], 0, 0)),
            pl.BlockSpec((1, d, f_dim), lambda b, be: (be[b], 0, 0)),
            pl.BlockSpec((1, 1, d), lambda b, be: (be[b], 0, 0)),
        ],
        out_specs=pl.BlockSpec((BTS, d), lambda b, be: (b, 0)),
    )
    return pl.pallas_call(
        _ffn_kernel,
        grid_spec=grid_spec,
        out_shape=jax.ShapeDtypeStruct((NROWS, d), jnp.float32),
    )(be, xs, w1, b1r, w2, b2r)


def _combine_kernel(y0_ref, y1_ref, p0_ref, p1_ref, out_ref):
    out_ref[...] = p0_ref[...] * y0_ref[0] + p1_ref[...] * y1_ref[0]


def _combine(yp, p0, p1):
    _, T, d = yp.shape
    BT = 256
    return pl.pallas_call(
        _combine_kernel,
        grid=(T // BT,),
        in_specs=[
            pl.BlockSpec((1, BT, d), lambda t: (0, t, 0)),
            pl.BlockSpec((1, BT, d), lambda t: (1, t, 0)),
            pl.BlockSpec((BT, 1), lambda t: (t, 0)),
            pl.BlockSpec((BT, 1), lambda t: (t, 0)),
        ],
        out_specs=pl.BlockSpec((BT, d), lambda t: (t, 0)),
        out_shape=jax.ShapeDtypeStruct((T, d), jnp.float32),
    )(yp, yp, p0, p1)


def kernel(x, gate_w, w1, b1, w2, b2):
    b, s, d = x.shape
    xf = x.reshape(-1, d)
    n_exp, f_dim = w1.shape[0], w1.shape[1]

    d0, d1, p0, p1, be = _route(xf, gate_w)
    d0f = d0.reshape(-1)
    d1f = d1.reshape(-1)
    be1 = be.reshape(-1)

    xs = _sc_scatter(xf, d0f, d1f)

    b1r = b1.reshape(n_exp, 1, f_dim)
    b2r = b2.reshape(n_exp, 1, d)
    ys = _ffn(xs, be1, w1, b1r, w2, b2r)

    yp = _sc_gather(ys, d0f, d1f)
    out = _combine(yp, p0, p1)
    return out.reshape(b, s, d)


# speedup vs baseline: 2.7607x; 1.0413x over previous
import functools

import jax
import jax.numpy as jnp
from jax import lax
from jax.experimental import pallas as pl
from jax.experimental.pallas import tpu as pltpu
from jax.experimental.pallas import tpu_sc as plsc

NEXP = 8
BTS = 256
NROWS = 5888
NBLK = NROWS // BTS


def _router_kernel(x_ref, gate_ref, d0_ref, d1_ref, p0_ref, p1_ref, be_ref):
    x = x_ref[...]
    T = x.shape[0]
    s = lax.dot_general(x, gate_ref[...], (((1,), (1,)), ((), ())),
                        preferred_element_type=jnp.float32)
    cols = lax.broadcasted_iota(jnp.int32, s.shape, 1)
    idx1 = jnp.argmax(s, axis=1)
    oh1 = (cols == idx1[:, None])
    m1 = jnp.max(s, axis=1, keepdims=True)
    s2 = jnp.where(oh1, -jnp.inf, s)
    idx2 = jnp.argmax(s2, axis=1)
    oh2 = (cols == idx2[:, None])
    m2 = jnp.max(s2, axis=1, keepdims=True)
    e2 = jnp.exp(m2 - m1)
    z = 1.0 + e2
    p0_ref[...] = 1.0 / z
    p1_ref[...] = e2 / z

    oh1f = oh1.astype(jnp.float32)
    oh2f = oh2.astype(jnp.float32)
    ohp = jnp.concatenate([oh1f, oh2f], axis=0)

    n = 2 * T
    acc = ohp
    k = 1
    while k < n:
        shifted = jnp.concatenate(
            [jnp.zeros((k, NEXP), jnp.float32), acc[:-k]], axis=0)
        acc = acc + shifted
        k *= 2
    excl = acc - ohp
    counts = acc[n - 1:n, :]

    pc = jnp.floor((counts + (BTS - 1)) * (1.0 / BTS)) * BTS
    end = pc
    k = 1
    while k < NEXP:
        end = end + jnp.concatenate(
            [jnp.zeros((1, k), jnp.float32), end[:, :-k]], axis=1)
        k *= 2
    off = end - pc

    offb = jnp.broadcast_to(off, (T, NEXP))
    d0 = jnp.sum(oh1f * (excl[:T] + offb), axis=1, keepdims=True)
    d1 = jnp.sum(oh2f * (excl[T:] + offb), axis=1, keepdims=True)
    d0_ref[...] = d0.astype(jnp.int32)
    d1_ref[...] = d1.astype(jnp.int32)

    jrow = (lax.broadcasted_iota(jnp.int32, (NBLK, NEXP), 0)
            .astype(jnp.float32) * float(BTS))
    endb = jnp.broadcast_to(end, (NBLK, NEXP))
    be = jnp.sum((jrow >= endb).astype(jnp.float32), axis=1, keepdims=True)
    be_ref[...] = jnp.clip(be, 0.0, float(NEXP - 1)).astype(jnp.int32)


def _route(xf, gate_w):
    T, d = xf.shape
    return pl.pallas_call(
        _router_kernel,
        grid=(1,),
        in_specs=[
            pl.BlockSpec((T, d), lambda i: (0, 0)),
            pl.BlockSpec((NEXP, d), lambda i: (0, 0)),
        ],
        out_specs=[
            pl.BlockSpec((T, 1), lambda i: (0, 0)),
            pl.BlockSpec((T, 1), lambda i: (0, 0)),
            pl.BlockSpec((T, 1), lambda i: (0, 0)),
            pl.BlockSpec((T, 1), lambda i: (0, 0)),
            pl.BlockSpec((NBLK, 1), lambda i: (0, 0)),
        ],
        out_shape=[
            jax.ShapeDtypeStruct((T, 1), jnp.int32),
            jax.ShapeDtypeStruct((T, 1), jnp.int32),
            jax.ShapeDtypeStruct((T, 1), jnp.float32),
            jax.ShapeDtypeStruct((T, 1), jnp.float32),
            jax.ShapeDtypeStruct((NBLK, 1), jnp.int32),
        ],
    )(xf, gate_w)


def _sc_scatter(xf, d0, d1):
    T, d = xf.shape
    info = plsc.get_sparse_core_info()
    nc, ns = info.num_cores, info.num_subcores
    nw = nc * ns
    tok_per_w = T // nw

    @functools.partial(
        pl.kernel,
        mesh=plsc.VectorSubcoreMesh(core_axis_name="c", subcore_axis_name="s"),
        out_type=jax.ShapeDtypeStruct((NROWS, d), jnp.float32),
        scratch_types=[
            pltpu.VMEM((tok_per_w,), jnp.int32),
            pltpu.VMEM((tok_per_w,), jnp.int32),
            pltpu.VMEM((tok_per_w, d), jnp.float32),
            pltpu.SemaphoreType.DMA,
            pltpu.SemaphoreType.DMA,
        ],
    )
    def k(x_hbm, d0_hbm, d1_hbm, xs_hbm, idx0_v, idx1_v, rows_v, sem0, sem1):
        wid = lax.axis_index("s") * nc + lax.axis_index("c")
        base = wid * tok_per_w
        pltpu.sync_copy(d0_hbm.at[pl.ds(base, tok_per_w)], idx0_v)
        pltpu.sync_copy(d1_hbm.at[pl.ds(base, tok_per_w)], idx1_v)
        pltpu.sync_copy(x_hbm.at[pl.ds(base, tok_per_w)], rows_v)
        c0 = pltpu.async_copy(rows_v, xs_hbm.at[idx0_v], sem0)
        c1 = pltpu.async_copy(rows_v, xs_hbm.at[idx1_v], sem1)
        c0.wait()
        c1.wait()

    return k(xf, d0, d1)


def _sc_combine(ys, d0, d1, p0f, p1f):
    _, d = ys.shape
    T = d0.shape[0]
    info = plsc.get_sparse_core_info()
    nc, ns = info.num_cores, info.num_subcores
    nw = nc * ns
    L = info.num_lanes
    tok_per_w = T // nw
    nch = d // L

    @functools.partial(
        pl.kernel,
        mesh=plsc.VectorSubcoreMesh(core_axis_name="c", subcore_axis_name="s"),
        compiler_params=pltpu.CompilerParams(needs_layout_passes=False),
        out_type=jax.ShapeDtypeStruct((T, d), jnp.float32),
        scratch_types=[
            pltpu.VMEM((tok_per_w,), jnp.int32),
            pltpu.VMEM((tok_per_w,), jnp.int32),
            pltpu.VMEM((tok_per_w,), jnp.float32),
            pltpu.VMEM((tok_per_w,), jnp.float32),
            pltpu.VMEM((tok_per_w, d), jnp.float32),
            pltpu.VMEM((tok_per_w, d), jnp.float32),
            pltpu.SemaphoreType.DMA,
            pltpu.SemaphoreType.DMA,
        ],
    )
    def k(ys_hbm, d0_hbm, d1_hbm, p0_hbm, p1_hbm, out_hbm,
          idx0_v, idx1_v, p0_v, p1_v, r0_v, r1_v, sem0, sem1):
        wid = lax.axis_index("s") * nc + lax.axis_index("c")
        base = wid * tok_per_w
        pltpu.sync_copy(d0_hbm.at[pl.ds(base, tok_per_w)], idx0_v)
        pltpu.sync_copy(d1_hbm.at[pl.ds(base, tok_per_w)], idx1_v)
        pltpu.sync_copy(p0_hbm.at[pl.ds(base, tok_per_w)], p0_v)
        pltpu.sync_copy(p1_hbm.at[pl.ds(base, tok_per_w)], p1_v)
        c0 = pltpu.async_copy(ys_hbm.at[idx0_v], r0_v, sem0)
        c1 = pltpu.async_copy(ys_hbm.at[idx1_v], r1_v, sem1)
        c0.wait()
        c1.wait()

        def tok_body(i, carry):
            bcast = lax.broadcasted_iota(jnp.int32, (L,), 0) * 0 + i
            pb0 = plsc.load_gather(p0_v, [bcast])
            pb1 = plsc.load_gather(p1_v, [bcast])
            for c in range(nch):
                a = r0_v[i, pl.ds(c * L, L)]
                bb = r1_v[i, pl.ds(c * L, L)]
                r0_v[i, pl.ds(c * L, L)] = pb0 * a + pb1 * bb
            return carry

        lax.fori_loop(0, tok_per_w, tok_body, 0)
        pltpu.sync_copy(r0_v, out_hbm.at[pl.ds(base, tok_per_w)])

    return k(ys, d0, d1, p0f, p1f)


def _ffn_kernel(be_ref, xs_ref, w1_ref, b1_ref, w2_ref, b2_ref, out_ref):
    xb = xs_ref[...]
    w1c = w1_ref[0]
    h = lax.dot_general(xb, w1c, (((1,), (1,)), ((), ())),
                        preferred_element_type=jnp.float32)
    h = h + b1_ref[0]
    h = 0.5 * h * (1.0 + lax.erf(h * 0.7071067811865476))
    w2c = w2_ref[0]
    y = lax.dot_general(h, w2c, (((1,), (1,)), ((), ())),
                        preferred_element_type=jnp.float32)
    out_ref[...] = y + b2_ref[0]


def _ffn(xs, be, w1, b1r, w2, b2r):
    d = xs.shape[1]
    f_dim = w1.shape[1]
    grid_spec = pltpu.PrefetchScalarGridSpec(
        num_scalar_prefetch=1,
        grid=(NBLK,),
        in_specs=[
            pl.BlockSpec((BTS, d), lambda b, be: (b, 0)),
            pl.BlockSpec((1, f_dim, d), lambda b, be: (be[b], 0, 0)),
            pl.BlockSpec((1, 1, f_dim), lambda b, be: (be[b], 0, 0)),
            pl.BlockSpec((1, d, f_dim), lambda b, be: (be[b], 0, 0)),
            pl.BlockSpec((1, 1, d), lambda b, be: (be[b], 0, 0)),
        ],
        out_specs=pl.BlockSpec((BTS, d), lambda b, be: (b, 0)),
    )
    return pl.pallas_call(
        _ffn_kernel,
        grid_spec=grid_spec,
        out_shape=jax.ShapeDtypeStruct((NROWS, d), jnp.float32),
    )(be, xs, w1, b1r, w2, b2r)


def _combine_kernel(y0_ref, y1_ref, p0_ref, p1_ref, out_ref):
    out_ref[...] = p0_ref[...] * y0_ref[0] + p1_ref[...] * y1_ref[0]


def _combine(yp, p0, p1):
    _, T, d = yp.shape
    BT = 256
    return pl.pallas_call(
        _combine_kernel,
        grid=(T // BT,),
        in_specs=[
            pl.BlockSpec((1, BT, d), lambda t: (0, t, 0)),
            pl.BlockSpec((1, BT, d), lambda t: (1, t, 0)),
            pl.BlockSpec((BT, 1), lambda t: (t, 0)),
            pl.BlockSpec((BT, 1), lambda t: (t, 0)),
        ],
        out_specs=pl.BlockSpec((BT, d), lambda t: (t, 0)),
        out_shape=jax.ShapeDtypeStruct((T, d), jnp.float32),
    )(yp, yp, p0, p1)


def kernel(x, gate_w, w1, b1, w2, b2):
    b, s, d = x.shape
    xf = x.reshape(-1, d)
    n_exp, f_dim = w1.shape[0], w1.shape[1]

    d0, d1, p0, p1, be = _route(xf, gate_w)
    d0f = d0.reshape(-1)
    d1f = d1.reshape(-1)
    be1 = be.reshape(-1)

    xs = _sc_scatter(xf, d0f, d1f)

    b1r = b1.reshape(n_exp, 1, f_dim)
    b2r = b2.reshape(n_exp, 1, d)
    ys = _ffn(xs, be1, w1, b1r, w2, b2r)

    out = _sc_combine(ys, d0f, d1f, p0.reshape(-1), p1.reshape(-1))
    return out.reshape(b, s, d)
